# baseline (speedup 1.0000x reference)
import functools

import jax
import jax.numpy as jnp
import numpy as np
from jax import lax
from jax.experimental import pallas as pl
from jax.experimental.pallas import tpu as pltpu
from jax.experimental.pallas import tpu_sc as plsc

N_NODES = 10000
N_EDGES = 320000
DIM = 128

NC = 2
NS = 16
NW = NC * NS
E_PER_W = N_EDGES // NW
CHUNK = 80
SUB = CHUNK // 2
N_CHUNKS = E_PER_W // CHUNK
STRIPE = 80
N_STRIPES = N_NODES // STRIPE
STRIPES_PER_TILE = -(-N_STRIPES // NS)
LANES = 16

NBUF = 2
N_RING = (N_CHUNKS - 1) // NBUF

_PERM = np.zeros(DIM, np.int32)
for _g in range(DIM // 32):
    for _t in range(16):
        _PERM[32 * _g + 2 * _t] = 32 * _g + _t
        _PERM[32 * _g + 2 * _t + 1] = 32 * _g + 16 + _t


def _sc_edge_body(xpk_hbm, src_hbm, dst_hbm, ea_hbm, out_hbm,
                  src_bufs, dstl_bufs, dsth_bufs, xrow_bufs, ea_bufs,
                  msg_bufs, acc_sh, isems, esems, gsems, dsems, ssems):
    cid = lax.axis_index("c")
    sid = lax.axis_index("s")
    wid = sid * NC + cid

    def zero_row(r, _):
        for j in range(DIM // LANES):
            ea_bufs[0][r, pl.ds(j * LANES, LANES)] = (
                jnp.zeros((LANES,), jnp.float32))
        return 0
    lax.fori_loop(0, STRIPE, zero_row, 0)
    for k in range(STRIPES_PER_TILE):
        s = sid + k * NS
        @pl.when(s < N_STRIPES)
        def _():
            pltpu.sync_copy(ea_bufs[0], acc_sh.at[pl.ds(s * STRIPE, STRIPE)])
    plsc.subcore_barrier()

    base0 = wid * E_PER_W

    def start_feed(i, b):
        base = base0 + i * CHUNK
        pltpu.async_copy(src_hbm.at[pl.ds(base, CHUNK)], src_bufs[b], isems[b])
        pltpu.make_async_copy(
            src_hbm.at[pl.ds(0, CHUNK)], src_bufs[b], isems[b]).wait()
        pltpu.async_copy(xpk_hbm.at[src_bufs[b]], xrow_bufs[b], gsems[b])
        pltpu.async_copy(ea_hbm.at[pl.ds(base, CHUNK)], ea_bufs[b], esems[b])

    def start_dst(i, b):
        base = base0 + i * CHUNK
        pltpu.async_copy(dst_hbm.at[pl.ds(base, SUB)], dstl_bufs[b], dsems[b])
        pltpu.async_copy(dst_hbm.at[pl.ds(base + SUB, SUB)], dsth_bufs[b],
                         dsems[b])

    def wait_feed(b):
        pltpu.make_async_copy(
            xpk_hbm.at[src_bufs[b]], xrow_bufs[b], gsems[b]).wait()
        pltpu.make_async_copy(
            ea_hbm.at[pl.ds(0, CHUNK)], ea_bufs[b], esems[b]).wait()

    def wait_dst(b):
        pltpu.make_async_copy(
            dst_hbm.at[pl.ds(0, SUB)], dstl_bufs[b], dsems[b]).wait()
        pltpu.make_async_copy(
            dst_hbm.at[pl.ds(0, SUB)], dsth_bufs[b], dsems[b]).wait()

    def compute_sub(b, s, m):
        def row_body(r, _):
            rr = s * SUB + r
            for g in range(DIM // 32):
                xi = xrow_bufs[b][rr, pl.ds(g * 16, 16)]
                xb = plsc.bitcast(xi, jnp.bfloat16)
                xa, xc = plsc.unpack(xb, format=plsc.PackFormat.INTERLEAVED)
                lo = pl.ds(g * 32, 16)
                hi = pl.ds(g * 32 + 16, 16)
                msg_bufs[m][r, lo] = jnp.maximum(xa + ea_bufs[b][rr, lo], 0.0)
                msg_bufs[m][r, hi] = jnp.maximum(xc + ea_bufs[b][rr, hi], 0.0)
            return 0
        lax.fori_loop(0, SUB, row_body, 0)

    def body_fn(i, b, bn):
        @pl.when(i + 1 < N_CHUNKS)
        def _():
            start_feed(i + 1, bn)
        wait_feed(b)
        wait_dst(b)

        @pl.when(i > 0)
        def _():
            pltpu.make_async_copy(
                msg_bufs[0], acc_sh.at[dstl_bufs[b]], ssems[0]).wait()
        compute_sub(b, 0, 0)
        pltpu.async_copy(msg_bufs[0], acc_sh.at[dstl_bufs[b]], ssems[0],
                         add=True)

        @pl.when(i > 0)
        def _():
            pltpu.make_async_copy(
                msg_bufs[1], acc_sh.at[dsth_bufs[b]], ssems[1]).wait()
        compute_sub(b, 1, 1)
        pltpu.async_copy(msg_bufs[1], acc_sh.at[dsth_bufs[b]], ssems[1],
                         add=True)

        @pl.when(i + 1 < N_CHUNKS)
        def _():
            start_dst(i + 1, bn)

    start_feed(0, 0)
    start_dst(0, 0)

    def ring_body(g, _):
        i0 = g * NBUF
        for j in range(NBUF):
            body_fn(i0 + j, j, (j + 1) % NBUF)
        return 0

    lax.fori_loop(0, N_RING, ring_body, 0)
    body_fn(jnp.int32(N_CHUNKS - 1), 0, 1)
    pltpu.make_async_copy(msg_bufs[0], acc_sh.at[dstl_bufs[0]], ssems[0]).wait()
    pltpu.make_async_copy(msg_bufs[1], acc_sh.at[dsth_bufs[0]], ssems[1]).wait()
    plsc.subcore_barrier()

    for k in range(STRIPES_PER_TILE):
        s = sid + k * NS
        @pl.when(s < N_STRIPES)
        def _():
            pltpu.sync_copy(acc_sh.at[pl.ds(s * STRIPE, STRIPE)],
                            out_hbm.at[cid, pl.ds(s * STRIPE, STRIPE)])


_sc_edge = functools.partial(
    pl.kernel,
    out_type=jax.ShapeDtypeStruct((NC, N_NODES, DIM), jnp.float32),
    mesh=plsc.VectorSubcoreMesh(core_axis_name="c", subcore_axis_name="s",
                                num_cores=NC, num_subcores=NS),
    compiler_params=pltpu.CompilerParams(needs_layout_passes=False,
                                         use_tc_tiling_on_sc=False),
    scratch_types=[
        tuple(pltpu.VMEM((CHUNK,), jnp.int32) for _ in range(NBUF)),
        tuple(pltpu.VMEM((SUB,), jnp.int32) for _ in range(NBUF)),
        tuple(pltpu.VMEM((SUB,), jnp.int32) for _ in range(NBUF)),
        tuple(pltpu.VMEM((CHUNK, DIM // 2), jnp.int32) for _ in range(NBUF)),
        tuple(pltpu.VMEM((CHUNK, DIM), jnp.float32) for _ in range(NBUF)),
        tuple(pltpu.VMEM((SUB, DIM), jnp.float32) for _ in range(NBUF)),
        pltpu.VMEM_SHARED((N_NODES, DIM), jnp.float32),
        tuple(pltpu.SemaphoreType.DMA for _ in range(NBUF)),
        tuple(pltpu.SemaphoreType.DMA for _ in range(NBUF)),
        tuple(pltpu.SemaphoreType.DMA for _ in range(NBUF)),
        tuple(pltpu.SemaphoreType.DMA for _ in range(NBUF)),
        tuple(pltpu.SemaphoreType.DMA for _ in range(NBUF)),
    ],
)(_sc_edge_body)


def _tc_mlp_body(eps_ref, x_ref, p_ref, w1_ref, b1_ref, g_ref, be_ref,
                 w2_ref, b2_ref, o_ref):
    h = x_ref[...] * (1.0 + eps_ref[0]) + p_ref[0] + p_ref[1]
    h1 = lax.dot_general(h, w1_ref[...], (((1,), (1,)), ((), ())),
                         preferred_element_type=jnp.float32) + b1_ref[...]
    mean = jnp.mean(h1, axis=0, keepdims=True)
    var = jnp.mean(jnp.square(h1 - mean), axis=0, keepdims=True)
    h2 = (h1 - mean) * lax.rsqrt(var + 1e-5) * g_ref[...] + be_ref[...]
    h2 = jnp.maximum(h2, 0.0)
    o_ref[...] = lax.dot_general(h2, w2_ref[...], (((1,), (1,)), ((), ())),
                                 preferred_element_type=jnp.float32) + b2_ref[...]


def _tc_mlp(eps, x, partials, w1, b1, gamma, beta, w2, b2):
    return pl.pallas_call(
        _tc_mlp_body,
        out_shape=jax.ShapeDtypeStruct((N_NODES, DIM), jnp.float32),
        in_specs=[
            pl.BlockSpec(memory_space=pltpu.SMEM),
            pl.BlockSpec(memory_space=pltpu.VMEM),
            pl.BlockSpec(memory_space=pltpu.VMEM),
            pl.BlockSpec(memory_space=pltpu.VMEM),
            pl.BlockSpec(memory_space=pltpu.VMEM),
            pl.BlockSpec(memory_space=pltpu.VMEM),
            pl.BlockSpec(memory_space=pltpu.VMEM),
            pl.BlockSpec(memory_space=pltpu.VMEM),
            pl.BlockSpec(memory_space=pltpu.VMEM),
        ],
        out_specs=pl.BlockSpec(memory_space=pltpu.VMEM),
    )(eps, x, partials, w1, b1, gamma, beta, w2, b2)


def kernel(x, edge_index, edge_attr, eps, W1, b1, gamma, beta, W2, b2):
    dst = edge_index[0]
    src = edge_index[1]
    x_sw = x[:, _PERM].astype(jnp.bfloat16)
    x_pk = lax.bitcast_convert_type(
        x_sw.reshape(N_NODES, DIM // 2, 2), jnp.int32)
    partials = _sc_edge(x_pk, src, dst, edge_attr)
    return _tc_mlp(eps, x, partials, W1,
                   b1.reshape(1, DIM), gamma.reshape(1, DIM),
                   beta.reshape(1, DIM), W2, b2.reshape(1, DIM))


# speedup vs baseline: 1.5581x; 1.5581x over previous
import functools

import jax
import jax.numpy as jnp
from jax import lax
from jax.experimental import pallas as pl
from jax.experimental.pallas import tpu as pltpu
from jax.experimental.pallas import tpu_sc as plsc

N_NODES = 10000
N_EDGES = 320000
DIM = 128

NC = 2
NS = 16
NW = NC * NS
E_PER_W = N_EDGES // NW
CHUNK = 80
N_CHUNKS = E_PER_W // CHUNK
STRIPE = 80
N_STRIPES = N_NODES // STRIPE
STRIPES_PER_TILE = -(-N_STRIPES // NS)
LANES = 16


NBUF = 2
N_RING = (N_CHUNKS - 1) // NBUF
UNROLL = 2


SUB = CHUNK // 2


def _sc_edge_body(x_hbm, src_hbm, dst_hbm, ea_hbm, out_hbm,
                  src_bufs, dstl_bufs, dsth_bufs, rows_bufs, msg_bufs,
                  acc_sh, isems, dsems, gsems, esems, ssems):
    cid = lax.axis_index("c")
    sid = lax.axis_index("s")
    wid = sid * NC + cid

    def zero_row(r, _):
        for j in range(DIM // LANES):
            rows_bufs[0][r, pl.ds(j * LANES, LANES)] = (
                jnp.zeros((LANES,), jnp.float32))
        return 0
    lax.fori_loop(0, STRIPE, zero_row, 0)
    for k in range(STRIPES_PER_TILE):
        s = sid + k * NS
        @pl.when(s < N_STRIPES)
        def _():
            pltpu.sync_copy(rows_bufs[0], acc_sh.at[pl.ds(s * STRIPE, STRIPE)])
    plsc.subcore_barrier()

    base0 = wid * E_PER_W

    def start_src(i, b):
        base = base0 + i * CHUNK
        pltpu.async_copy(src_hbm.at[pl.ds(base, CHUNK)], src_bufs[b], isems[b])

    def start_dst(i, b):
        base = base0 + i * CHUNK
        pltpu.async_copy(dst_hbm.at[pl.ds(base, SUB)], dstl_bufs[b], dsems[b])
        pltpu.async_copy(dst_hbm.at[pl.ds(base + SUB, SUB)], dsth_bufs[b],
                         dsems[b])

    def start_ea(i, b, m):
        base = base0 + i * CHUNK + m * SUB
        pltpu.async_copy(ea_hbm.at[pl.ds(base, SUB)], msg_bufs[2 * b + m],
                         esems[2 * b + m])

    def start_gather(b):
        pltpu.async_copy(x_hbm.at[src_bufs[b]], rows_bufs[b], gsems[b])

    def wait_scatter(mb):
        pltpu.make_async_copy(
            msg_bufs[mb], acc_sh.at[dstl_bufs[0]], ssems[mb]).wait()

    def compute_sub(b, m):
        mb = 2 * b + m

        def row_body(r, _):
            rr = m * SUB + r
            for j in range(DIM // LANES):
                sl = pl.ds(j * LANES, LANES)
                msg_bufs[mb][r, sl] = jnp.maximum(
                    msg_bufs[mb][r, sl] + rows_bufs[b][rr, sl], 0.0)
            return 0
        lax.fori_loop(0, SUB, row_body, 0)

    def body_fn(i, b, bn):
        pltpu.make_async_copy(
            x_hbm.at[src_bufs[b]], rows_bufs[b], gsems[b]).wait()
        for m in range(2):
            pltpu.make_async_copy(
                ea_hbm.at[pl.ds(0, SUB)], msg_bufs[2 * b + m],
                esems[2 * b + m]).wait()

        @pl.when(i + 2 < N_CHUNKS)
        def _():
            start_src(i + 2, b)

        @pl.when(i + 1 < N_CHUNKS)
        def _():
            pltpu.make_async_copy(
                src_hbm.at[pl.ds(0, CHUNK)], src_bufs[bn], isems[bn]).wait()
            start_gather(bn)

        for _m in range(2):
            pltpu.make_async_copy(
                dst_hbm.at[pl.ds(0, SUB)], dstl_bufs[b], dsems[b]).wait()

        @pl.when(i + 1 < N_CHUNKS)
        def _():
            @pl.when(i > 0)
            def _():
                wait_scatter(2 * bn)
                wait_scatter(2 * bn + 1)
            start_ea(i + 1, bn, 0)
            start_ea(i + 1, bn, 1)
            start_dst(i + 1, bn)

        compute_sub(b, 0)
        pltpu.async_copy(msg_bufs[2 * b], acc_sh.at[dstl_bufs[b]],
                         ssems[2 * b], add=True)
        compute_sub(b, 1)
        pltpu.async_copy(msg_bufs[2 * b + 1], acc_sh.at[dsth_bufs[b]],
                         ssems[2 * b + 1], add=True)

    start_src(0, 0)
    start_dst(0, 0)
    pltpu.make_async_copy(
        src_hbm.at[pl.ds(0, CHUNK)], src_bufs[0], isems[0]).wait()
    start_gather(0)
    start_ea(0, 0, 0)
    start_ea(0, 0, 1)
    start_src(1, 1)

    def ring_body(g, _):
        i0 = g * NBUF
        for j in range(NBUF):
            body_fn(i0 + j, j, (j + 1) % NBUF)
        return 0

    lax.fori_loop(0, N_RING, ring_body, 0)
    body_fn(jnp.int32(N_CHUNKS - 1), 0, 1)
    for mb in range(4):
        wait_scatter(mb)
    plsc.subcore_barrier()

    for k in range(STRIPES_PER_TILE):
        s = sid + k * NS
        @pl.when(s < N_STRIPES)
        def _():
            pltpu.sync_copy(acc_sh.at[pl.ds(s * STRIPE, STRIPE)],
                            out_hbm.at[cid, pl.ds(s * STRIPE, STRIPE)])


_sc_edge = functools.partial(
    pl.kernel,
    out_type=jax.ShapeDtypeStruct((NC, N_NODES, DIM), jnp.float32),
    mesh=plsc.VectorSubcoreMesh(core_axis_name="c", subcore_axis_name="s",
                                num_cores=NC, num_subcores=NS),
    scratch_types=[
        tuple(pltpu.VMEM((CHUNK,), jnp.int32) for _ in range(NBUF)),
        tuple(pltpu.VMEM((SUB,), jnp.int32) for _ in range(NBUF)),
        tuple(pltpu.VMEM((SUB,), jnp.int32) for _ in range(NBUF)),
        tuple(pltpu.VMEM((CHUNK, DIM), jnp.float32) for _ in range(NBUF)),
        tuple(pltpu.VMEM((SUB, DIM), jnp.float32) for _ in range(4)),
        pltpu.VMEM_SHARED((N_NODES, DIM), jnp.float32),
        tuple(pltpu.SemaphoreType.DMA for _ in range(NBUF)),
        tuple(pltpu.SemaphoreType.DMA for _ in range(NBUF)),
        tuple(pltpu.SemaphoreType.DMA for _ in range(NBUF)),
        tuple(pltpu.SemaphoreType.DMA for _ in range(4)),
        tuple(pltpu.SemaphoreType.DMA for _ in range(4)),
    ],
)(_sc_edge_body)


def _tc_mlp_body(eps_ref, x_ref, p_ref, w1_ref, b1_ref, g_ref, be_ref,
                 w2_ref, b2_ref, o_ref):
    h = x_ref[...] * (1.0 + eps_ref[0]) + p_ref[0] + p_ref[1]
    h1 = lax.dot_general(h, w1_ref[...], (((1,), (1,)), ((), ())),
                         preferred_element_type=jnp.float32) + b1_ref[...]
    mean = jnp.mean(h1, axis=0, keepdims=True)
    var = jnp.mean(jnp.square(h1 - mean), axis=0, keepdims=True)
    h2 = (h1 - mean) * lax.rsqrt(var + 1e-5) * g_ref[...] + be_ref[...]
    h2 = jnp.maximum(h2, 0.0)
    o_ref[...] = lax.dot_general(h2, w2_ref[...], (((1,), (1,)), ((), ())),
                                 preferred_element_type=jnp.float32) + b2_ref[...]


def _tc_mlp(eps, x, partials, w1, b1, gamma, beta, w2, b2):
    return pl.pallas_call(
        _tc_mlp_body,
        out_shape=jax.ShapeDtypeStruct((N_NODES, DIM), jnp.float32),
        in_specs=[
            pl.BlockSpec(memory_space=pltpu.SMEM),
            pl.BlockSpec(memory_space=pltpu.VMEM),
            pl.BlockSpec(memory_space=pltpu.VMEM),
            pl.BlockSpec(memory_space=pltpu.VMEM),
            pl.BlockSpec(memory_space=pltpu.VMEM),
            pl.BlockSpec(memory_space=pltpu.VMEM),
            pl.BlockSpec(memory_space=pltpu.VMEM),
            pl.BlockSpec(memory_space=pltpu.VMEM),
            pl.BlockSpec(memory_space=pltpu.VMEM),
        ],
        out_specs=pl.BlockSpec(memory_space=pltpu.VMEM),
    )(eps, x, partials, w1, b1, gamma, beta, w2, b2)


def kernel(x, edge_index, edge_attr, eps, W1, b1, gamma, beta, W2, b2):
    dst = edge_index[0]
    src = edge_index[1]
    partials = _sc_edge(x, src, dst, edge_attr)
    return _tc_mlp(eps, x, partials, W1,
                   b1.reshape(1, DIM), gamma.reshape(1, DIM),
                   beta.reshape(1, DIM), W2, b2.reshape(1, DIM))


# speedup vs baseline: 1.5583x; 1.0001x over previous
import functools

import jax
import jax.numpy as jnp
from jax import lax
from jax.experimental import pallas as pl
from jax.experimental.pallas import tpu as pltpu
from jax.experimental.pallas import tpu_sc as plsc

N_NODES = 10000
N_EDGES = 320000
DIM = 128

NC = 2
NS = 16
NW = NC * NS
E_PER_W = N_EDGES // NW
CHUNK = 80
N_CHUNKS = E_PER_W // CHUNK
STRIPE = 80
N_STRIPES = N_NODES // STRIPE
STRIPES_PER_TILE = -(-N_STRIPES // NS)
LANES = 16


NBUF = 2
N_RING = (N_CHUNKS - 1) // NBUF
UNROLL = 2


SUB = CHUNK // 2


def _sc_edge_body(x_hbm, src_hbm, dst_hbm, ea_hbm, out_hbm,
                  src_bufs, dstl_bufs, dsth_bufs, rows_bufs, msg_bufs,
                  acc_sh, isems, dsems, gsems, esems, ssems):
    cid = lax.axis_index("c")
    sid = lax.axis_index("s")
    wid = sid * NC + cid
    base0 = wid * E_PER_W

    def start_src(i, b):
        base = base0 + i * CHUNK
        pltpu.async_copy(src_hbm.at[pl.ds(base, CHUNK)], src_bufs[b], isems[b])

    def start_dst(i, b):
        base = base0 + i * CHUNK
        pltpu.async_copy(dst_hbm.at[pl.ds(base, SUB)], dstl_bufs[b], dsems[b])
        pltpu.async_copy(dst_hbm.at[pl.ds(base + SUB, SUB)], dsth_bufs[b],
                         dsems[b])

    def start_ea(i, b, m):
        base = base0 + i * CHUNK + m * SUB
        pltpu.async_copy(ea_hbm.at[pl.ds(base, SUB)], msg_bufs[2 * b + m],
                         esems[2 * b + m])

    def start_gather(b):
        pltpu.async_copy(x_hbm.at[src_bufs[b]], rows_bufs[b], gsems[b])

    def wait_scatter(mb):
        pltpu.make_async_copy(
            msg_bufs[mb], acc_sh.at[dstl_bufs[0]], ssems[mb]).wait()

    def compute_sub(b, m):
        mb = 2 * b + m

        def row_body(r, _):
            rr = m * SUB + r
            for j in range(DIM // LANES):
                sl = pl.ds(j * LANES, LANES)
                msg_bufs[mb][r, sl] = jnp.maximum(
                    msg_bufs[mb][r, sl] + rows_bufs[b][rr, sl], 0.0)
            return 0
        lax.fori_loop(0, SUB, row_body, 0)

    def body_fn(i, b, bn):
        pltpu.make_async_copy(
            x_hbm.at[src_bufs[b]], rows_bufs[b], gsems[b]).wait()
        for m in range(2):
            pltpu.make_async_copy(
                ea_hbm.at[pl.ds(0, SUB)], msg_bufs[2 * b + m],
                esems[2 * b + m]).wait()

        @pl.when(i + 2 < N_CHUNKS)
        def _():
            start_src(i + 2, b)

        @pl.when(i + 1 < N_CHUNKS)
        def _():
            pltpu.make_async_copy(
                src_hbm.at[pl.ds(0, CHUNK)], src_bufs[bn], isems[bn]).wait()
            start_gather(bn)

        for _m in range(2):
            pltpu.make_async_copy(
                dst_hbm.at[pl.ds(0, SUB)], dstl_bufs[b], dsems[b]).wait()

        @pl.when(i + 1 < N_CHUNKS)
        def _():
            @pl.when(i > 0)
            def _():
                wait_scatter(2 * bn)
                wait_scatter(2 * bn + 1)
            start_ea(i + 1, bn, 0)
            start_ea(i + 1, bn, 1)
            start_dst(i + 1, bn)

        compute_sub(b, 0)
        pltpu.async_copy(msg_bufs[2 * b], acc_sh.at[dstl_bufs[b]],
                         ssems[2 * b], add=True)
        compute_sub(b, 1)
        pltpu.async_copy(msg_bufs[2 * b + 1], acc_sh.at[dsth_bufs[b]],
                         ssems[2 * b + 1], add=True)

    start_src(0, 0)
    start_dst(0, 0)
    start_ea(0, 0, 0)
    start_ea(0, 0, 1)
    start_src(1, 1)

    def zero_row(r, _):
        for j in range(DIM // LANES):
            rows_bufs[0][r, pl.ds(j * LANES, LANES)] = (
                jnp.zeros((LANES,), jnp.float32))
        return 0
    lax.fori_loop(0, STRIPE, zero_row, 0)
    for k in range(STRIPES_PER_TILE):
        s = sid + k * NS
        @pl.when(s < N_STRIPES)
        def _():
            pltpu.sync_copy(rows_bufs[0], acc_sh.at[pl.ds(s * STRIPE, STRIPE)])
    plsc.subcore_barrier()

    pltpu.make_async_copy(
        src_hbm.at[pl.ds(0, CHUNK)], src_bufs[0], isems[0]).wait()
    start_gather(0)

    def ring_body(g, _):
        i0 = g * NBUF
        for j in range(NBUF):
            body_fn(i0 + j, j, (j + 1) % NBUF)
        return 0

    lax.fori_loop(0, N_RING, ring_body, 0)
    body_fn(jnp.int32(N_CHUNKS - 1), 0, 1)
    for mb in range(4):
        wait_scatter(mb)
    plsc.subcore_barrier()

    for k in range(STRIPES_PER_TILE):
        s = sid + k * NS
        @pl.when(s < N_STRIPES)
        def _():
            pltpu.sync_copy(acc_sh.at[pl.ds(s * STRIPE, STRIPE)],
                            out_hbm.at[cid, pl.ds(s * STRIPE, STRIPE)])


_sc_edge = functools.partial(
    pl.kernel,
    out_type=jax.ShapeDtypeStruct((NC, N_NODES, DIM), jnp.float32),
    mesh=plsc.VectorSubcoreMesh(core_axis_name="c", subcore_axis_name="s",
                                num_cores=NC, num_subcores=NS),
    scratch_types=[
        tuple(pltpu.VMEM((CHUNK,), jnp.int32) for _ in range(NBUF)),
        tuple(pltpu.VMEM((SUB,), jnp.int32) for _ in range(NBUF)),
        tuple(pltpu.VMEM((SUB,), jnp.int32) for _ in range(NBUF)),
        tuple(pltpu.VMEM((CHUNK, DIM), jnp.float32) for _ in range(NBUF)),
        tuple(pltpu.VMEM((SUB, DIM), jnp.float32) for _ in range(4)),
        pltpu.VMEM_SHARED((N_NODES, DIM), jnp.float32),
        tuple(pltpu.SemaphoreType.DMA for _ in range(NBUF)),
        tuple(pltpu.SemaphoreType.DMA for _ in range(NBUF)),
        tuple(pltpu.SemaphoreType.DMA for _ in range(NBUF)),
        tuple(pltpu.SemaphoreType.DMA for _ in range(4)),
        tuple(pltpu.SemaphoreType.DMA for _ in range(4)),
    ],
)(_sc_edge_body)


def _tc_mlp_body(eps_ref, x_ref, p_ref, w1_ref, b1_ref, g_ref, be_ref,
                 w2_ref, b2_ref, o_ref):
    h = x_ref[...] * (1.0 + eps_ref[0]) + p_ref[0] + p_ref[1]
    h1 = lax.dot_general(h, w1_ref[...], (((1,), (1,)), ((), ())),
                         preferred_element_type=jnp.float32) + b1_ref[...]
    mean = jnp.mean(h1, axis=0, keepdims=True)
    var = jnp.mean(jnp.square(h1 - mean), axis=0, keepdims=True)
    h2 = (h1 - mean) * lax.rsqrt(var + 1e-5) * g_ref[...] + be_ref[...]
    h2 = jnp.maximum(h2, 0.0)
    o_ref[...] = lax.dot_general(h2, w2_ref[...], (((1,), (1,)), ((), ())),
                                 preferred_element_type=jnp.float32) + b2_ref[...]


def _tc_mlp(eps, x, partials, w1, b1, gamma, beta, w2, b2):
    return pl.pallas_call(
        _tc_mlp_body,
        out_shape=jax.ShapeDtypeStruct((N_NODES, DIM), jnp.float32),
        in_specs=[
            pl.BlockSpec(memory_space=pltpu.SMEM),
            pl.BlockSpec(memory_space=pltpu.VMEM),
            pl.BlockSpec(memory_space=pltpu.VMEM),
            pl.BlockSpec(memory_space=pltpu.VMEM),
            pl.BlockSpec(memory_space=pltpu.VMEM),
            pl.BlockSpec(memory_space=pltpu.VMEM),
            pl.BlockSpec(memory_space=pltpu.VMEM),
            pl.BlockSpec(memory_space=pltpu.VMEM),
            pl.BlockSpec(memory_space=pltpu.VMEM),
        ],
        out_specs=pl.BlockSpec(memory_space=pltpu.VMEM),
    )(eps, x, partials, w1, b1, gamma, beta, w2, b2)


def kernel(x, edge_index, edge_attr, eps, W1, b1, gamma, beta, W2, b2):
    dst = edge_index[0]
    src = edge_index[1]
    partials = _sc_edge(x, src, dst, edge_attr)
    return _tc_mlp(eps, x, partials, W1,
                   b1.reshape(1, DIM), gamma.reshape(1, DIM),
                   beta.reshape(1, DIM), W2, b2.reshape(1, DIM))


# speedup vs baseline: 1.5976x; 1.0252x over previous
import functools

import jax
import jax.numpy as jnp
from jax import lax
from jax.experimental import pallas as pl
from jax.experimental.pallas import tpu as pltpu
from jax.experimental.pallas import tpu_sc as plsc

N_NODES = 10000
N_EDGES = 320000
DIM = 128

NC = 2
NS = 16
NW = NC * NS
E_PER_W = N_EDGES // NW
CHUNK = 96
N_CHUNKS = E_PER_W // CHUNK
TAIL = E_PER_W - N_CHUNKS * CHUNK
STRIPE = 80
N_STRIPES = N_NODES // STRIPE
STRIPES_PER_TILE = -(-N_STRIPES // NS)
LANES = 16


NBUF = 2
N_RING = N_CHUNKS // NBUF


SUB = CHUNK // 2


def _sc_edge_body(x_hbm, src_hbm, dst_hbm, ea_hbm, out_hbm,
                  src_bufs, dstl_bufs, dsth_bufs, rows_bufs, msg_bufs,
                  tdst_buf, acc_sh, isems, dsems, gsems, esems, ssems):
    cid = lax.axis_index("c")
    sid = lax.axis_index("s")
    wid = sid * NC + cid
    base0 = wid * E_PER_W

    def start_src(i, b):
        base = base0 + i * CHUNK
        pltpu.async_copy(src_hbm.at[pl.ds(base, CHUNK)], src_bufs[b], isems[b])

    def start_dst(i, b):
        base = base0 + i * CHUNK
        pltpu.async_copy(dst_hbm.at[pl.ds(base, SUB)], dstl_bufs[b], dsems[b])
        pltpu.async_copy(dst_hbm.at[pl.ds(base + SUB, SUB)], dsth_bufs[b],
                         dsems[b])

    def start_ea(i, b, m):
        base = base0 + i * CHUNK + m * SUB
        pltpu.async_copy(ea_hbm.at[pl.ds(base, SUB)], msg_bufs[2 * b + m],
                         esems[2 * b + m])

    def start_gather(b):
        pltpu.async_copy(x_hbm.at[src_bufs[b]], rows_bufs[b], gsems[b])

    def wait_scatter(mb):
        pltpu.make_async_copy(
            msg_bufs[mb], acc_sh.at[dstl_bufs[0]], ssems[mb]).wait()

    def compute_sub(b, m):
        mb = 2 * b + m

        def row_body(r, _):
            rr = m * SUB + r
            for j in range(DIM // LANES):
                sl = pl.ds(j * LANES, LANES)
                msg_bufs[mb][r, sl] = jnp.maximum(
                    msg_bufs[mb][r, sl] + rows_bufs[b][rr, sl], 0.0)
            return 0
        lax.fori_loop(0, SUB, row_body, 0)

    def body_fn(i, b, bn):
        pltpu.make_async_copy(
            x_hbm.at[src_bufs[b]], rows_bufs[b], gsems[b]).wait()
        for m in range(2):
            pltpu.make_async_copy(
                ea_hbm.at[pl.ds(0, SUB)], msg_bufs[2 * b + m],
                esems[2 * b + m]).wait()

        @pl.when(i + 2 < N_CHUNKS)
        def _():
            start_src(i + 2, b)

        @pl.when(i + 1 < N_CHUNKS)
        def _():
            pltpu.make_async_copy(
                src_hbm.at[pl.ds(0, CHUNK)], src_bufs[bn], isems[bn]).wait()
            start_gather(bn)

        for _m in range(2):
            pltpu.make_async_copy(
                dst_hbm.at[pl.ds(0, SUB)], dstl_bufs[b], dsems[b]).wait()

        @pl.when(i + 1 < N_CHUNKS)
        def _():
            @pl.when(i > 0)
            def _():
                wait_scatter(2 * bn)
                wait_scatter(2 * bn + 1)
            start_ea(i + 1, bn, 0)
            start_ea(i + 1, bn, 1)
            start_dst(i + 1, bn)

        compute_sub(b, 0)
        pltpu.async_copy(msg_bufs[2 * b], acc_sh.at[dstl_bufs[b]],
                         ssems[2 * b], add=True)
        compute_sub(b, 1)
        pltpu.async_copy(msg_bufs[2 * b + 1], acc_sh.at[dsth_bufs[b]],
                         ssems[2 * b + 1], add=True)

    start_src(0, 0)
    start_dst(0, 0)
    start_ea(0, 0, 0)
    start_ea(0, 0, 1)
    start_src(1, 1)

    def zero_row(r, _):
        for j in range(DIM // LANES):
            rows_bufs[0][r, pl.ds(j * LANES, LANES)] = (
                jnp.zeros((LANES,), jnp.float32))
        return 0
    lax.fori_loop(0, STRIPE, zero_row, 0)
    for k in range(STRIPES_PER_TILE):
        s = sid + k * NS
        @pl.when(s < N_STRIPES)
        def _():
            pltpu.sync_copy(rows_bufs[0].at[pl.ds(0, STRIPE)],
                            acc_sh.at[pl.ds(s * STRIPE, STRIPE)])
    plsc.subcore_barrier()

    pltpu.make_async_copy(
        src_hbm.at[pl.ds(0, CHUNK)], src_bufs[0], isems[0]).wait()
    start_gather(0)

    def ring_body(g, _):
        i0 = g * NBUF
        for j in range(NBUF):
            body_fn(i0 + j, j, (j + 1) % NBUF)
        return 0

    lax.fori_loop(0, N_RING, ring_body, 0)
    wait_scatter(0)
    wait_scatter(1)
    tbase = base0 + N_CHUNKS * CHUNK
    pltpu.async_copy(src_hbm.at[pl.ds(tbase, TAIL)],
                     src_bufs[0].at[pl.ds(0, TAIL)], isems[0])
    pltpu.async_copy(dst_hbm.at[pl.ds(tbase, TAIL)], tdst_buf, dsems[0])
    pltpu.async_copy(ea_hbm.at[pl.ds(tbase, TAIL)],
                     msg_bufs[0].at[pl.ds(0, TAIL)], esems[0])
    pltpu.make_async_copy(src_hbm.at[pl.ds(0, TAIL)],
                          src_bufs[0].at[pl.ds(0, TAIL)], isems[0]).wait()
    pltpu.async_copy(x_hbm.at[src_bufs[0].at[pl.ds(0, TAIL)]],
                     rows_bufs[0].at[pl.ds(0, TAIL)], gsems[0])
    pltpu.make_async_copy(x_hbm.at[src_bufs[0].at[pl.ds(0, TAIL)]],
                          rows_bufs[0].at[pl.ds(0, TAIL)], gsems[0]).wait()
    pltpu.make_async_copy(ea_hbm.at[pl.ds(0, TAIL)],
                          msg_bufs[0].at[pl.ds(0, TAIL)], esems[0]).wait()

    def tail_row(r, _):
        for j in range(DIM // LANES):
            sl = pl.ds(j * LANES, LANES)
            msg_bufs[0][r, sl] = jnp.maximum(
                msg_bufs[0][r, sl] + rows_bufs[0][r, sl], 0.0)
        return 0
    lax.fori_loop(0, TAIL, tail_row, 0)
    pltpu.make_async_copy(dst_hbm.at[pl.ds(0, TAIL)], tdst_buf,
                          dsems[0]).wait()
    pltpu.async_copy(msg_bufs[0].at[pl.ds(0, TAIL)], acc_sh.at[tdst_buf],
                     ssems[0], add=True)
    pltpu.make_async_copy(msg_bufs[0].at[pl.ds(0, TAIL)],
                          acc_sh.at[tdst_buf], ssems[0]).wait()
    wait_scatter(2)
    wait_scatter(3)
    plsc.subcore_barrier()

    for k in range(STRIPES_PER_TILE):
        s = sid + k * NS
        @pl.when(s < N_STRIPES)
        def _():
            pltpu.sync_copy(acc_sh.at[pl.ds(s * STRIPE, STRIPE)],
                            out_hbm.at[cid, pl.ds(s * STRIPE, STRIPE)])


_sc_edge = functools.partial(
    pl.kernel,
    out_type=jax.ShapeDtypeStruct((NC, N_NODES, DIM), jnp.float32),
    mesh=plsc.VectorSubcoreMesh(core_axis_name="c", subcore_axis_name="s",
                                num_cores=NC, num_subcores=NS),
    scratch_types=[
        tuple(pltpu.VMEM((CHUNK,), jnp.int32) for _ in range(NBUF)),
        tuple(pltpu.VMEM((SUB,), jnp.int32) for _ in range(NBUF)),
        tuple(pltpu.VMEM((SUB,), jnp.int32) for _ in range(NBUF)),
        tuple(pltpu.VMEM((CHUNK, DIM), jnp.float32) for _ in range(NBUF)),
        tuple(pltpu.VMEM((SUB, DIM), jnp.float32) for _ in range(4)),
        pltpu.VMEM((TAIL,), jnp.int32),
        pltpu.VMEM_SHARED((N_NODES, DIM), jnp.float32),
        tuple(pltpu.SemaphoreType.DMA for _ in range(NBUF)),
        tuple(pltpu.SemaphoreType.DMA for _ in range(NBUF)),
        tuple(pltpu.SemaphoreType.DMA for _ in range(NBUF)),
        tuple(pltpu.SemaphoreType.DMA for _ in range(4)),
        tuple(pltpu.SemaphoreType.DMA for _ in range(4)),
    ],
)(_sc_edge_body)


def _tc_mlp_body(eps_ref, x_ref, p_ref, w1_ref, b1_ref, g_ref, be_ref,
                 w2_ref, b2_ref, o_ref):
    h = x_ref[...] * (1.0 + eps_ref[0]) + p_ref[0] + p_ref[1]
    h1 = lax.dot_general(h, w1_ref[...], (((1,), (1,)), ((), ())),
                         preferred_element_type=jnp.float32) + b1_ref[...]
    mean = jnp.mean(h1, axis=0, keepdims=True)
    var = jnp.mean(jnp.square(h1 - mean), axis=0, keepdims=True)
    h2 = (h1 - mean) * lax.rsqrt(var + 1e-5) * g_ref[...] + be_ref[...]
    h2 = jnp.maximum(h2, 0.0)
    o_ref[...] = lax.dot_general(h2, w2_ref[...], (((1,), (1,)), ((), ())),
                                 preferred_element_type=jnp.float32) + b2_ref[...]


def _tc_mlp(eps, x, partials, w1, b1, gamma, beta, w2, b2):
    return pl.pallas_call(
        _tc_mlp_body,
        out_shape=jax.ShapeDtypeStruct((N_NODES, DIM), jnp.float32),
        in_specs=[
            pl.BlockSpec(memory_space=pltpu.SMEM),
            pl.BlockSpec(memory_space=pltpu.VMEM),
            pl.BlockSpec(memory_space=pltpu.VMEM),
            pl.BlockSpec(memory_space=pltpu.VMEM),
            pl.BlockSpec(memory_space=pltpu.VMEM),
            pl.BlockSpec(memory_space=pltpu.VMEM),
            pl.BlockSpec(memory_space=pltpu.VMEM),
            pl.BlockSpec(memory_space=pltpu.VMEM),
            pl.BlockSpec(memory_space=pltpu.VMEM),
        ],
        out_specs=pl.BlockSpec(memory_space=pltpu.VMEM),
    )(eps, x, partials, w1, b1, gamma, beta, w2, b2)


def kernel(x, edge_index, edge_attr, eps, W1, b1, gamma, beta, W2, b2):
    dst = edge_index[0]
    src = edge_index[1]
    partials = _sc_edge(x, src, dst, edge_attr)
    return _tc_mlp(eps, x, partials, W1,
                   b1.reshape(1, DIM), gamma.reshape(1, DIM),
                   beta.reshape(1, DIM), W2, b2.reshape(1, DIM))


# speedup vs baseline: 1.6648x; 1.0421x over previous
import functools

import jax
import jax.numpy as jnp
from jax import lax
from jax.experimental import pallas as pl
from jax.experimental.pallas import tpu as pltpu
from jax.experimental.pallas import tpu_sc as plsc

N_NODES = 10000
N_EDGES = 320000
DIM = 128

NC = 2
NS = 16
NW = NC * NS
E_PER_W = N_EDGES // NW
CHUNK = 96
N_CHUNKS = E_PER_W // CHUNK
TAIL = E_PER_W - N_CHUNKS * CHUNK
STRIPE = 80
N_STRIPES = N_NODES // STRIPE
STRIPES_PER_TILE = -(-N_STRIPES // NS)
LANES = 16


NBUF = 2
N_RING = N_CHUNKS // NBUF


SUB = CHUNK // 2


def _sc_edge_body(x_hbm, src_hbm, dst_hbm, ea_hbm, out_hbm,
                  src_bufs, dstl_bufs, dsth_bufs, rows_bufs, msg_bufs,
                  tdst_buf, acc_sh, isems, dsems, gsems, esems, ssems):
    cid = lax.axis_index("c")
    sid = lax.axis_index("s")
    wid = sid * NC + cid
    base0 = wid * E_PER_W

    def start_src(i, b):
        base = base0 + i * CHUNK
        pltpu.async_copy(src_hbm.at[pl.ds(base, CHUNK)], src_bufs[b], isems[b])

    def start_dst(i, b):
        base = base0 + i * CHUNK
        pltpu.async_copy(dst_hbm.at[pl.ds(base, SUB)], dstl_bufs[b], dsems[b])
        pltpu.async_copy(dst_hbm.at[pl.ds(base + SUB, SUB)], dsth_bufs[b],
                         dsems[b])

    def start_ea(i, b):
        base = base0 + i * CHUNK
        pltpu.async_copy(ea_hbm.at[pl.ds(base, CHUNK)], msg_bufs[b], esems[b])

    def start_gather(b):
        pltpu.async_copy(x_hbm.at[src_bufs[b]], rows_bufs[b], gsems[b])

    def wait_scatter(mb):
        pltpu.make_async_copy(
            msg_bufs[mb // 2].at[pl.ds((mb % 2) * SUB, SUB)],
            acc_sh.at[dstl_bufs[0]], ssems[mb]).wait()

    def compute_sub(b, m):
        def row_body(r, _):
            rr = m * SUB + r
            for j in range(DIM // LANES):
                sl = pl.ds(j * LANES, LANES)
                msg_bufs[b][rr, sl] = jnp.maximum(
                    msg_bufs[b][rr, sl] + rows_bufs[b][rr, sl], 0.0)
            return 0
        lax.fori_loop(0, SUB, row_body, 0)

    def body_fn(i, b, bn):
        pltpu.make_async_copy(
            x_hbm.at[src_bufs[b]], rows_bufs[b], gsems[b]).wait()
        pltpu.make_async_copy(
            ea_hbm.at[pl.ds(0, CHUNK)], msg_bufs[b], esems[b]).wait()

        @pl.when(i + 2 < N_CHUNKS)
        def _():
            start_src(i + 2, b)

        @pl.when(i + 1 < N_CHUNKS)
        def _():
            pltpu.make_async_copy(
                src_hbm.at[pl.ds(0, CHUNK)], src_bufs[bn], isems[bn]).wait()
            start_gather(bn)

        for _m in range(2):
            pltpu.make_async_copy(
                dst_hbm.at[pl.ds(0, SUB)], dstl_bufs[b], dsems[b]).wait()

        @pl.when(i + 1 < N_CHUNKS)
        def _():
            @pl.when(i > 0)
            def _():
                wait_scatter(2 * bn)
                wait_scatter(2 * bn + 1)
            start_ea(i + 1, bn)
            start_dst(i + 1, bn)

        compute_sub(b, 0)
        pltpu.async_copy(msg_bufs[b].at[pl.ds(0, SUB)],
                         acc_sh.at[dstl_bufs[b]], ssems[2 * b], add=True)
        compute_sub(b, 1)
        pltpu.async_copy(msg_bufs[b].at[pl.ds(SUB, SUB)],
                         acc_sh.at[dsth_bufs[b]], ssems[2 * b + 1], add=True)

    start_src(0, 0)
    start_dst(0, 0)
    start_ea(0, 0)
    start_src(1, 1)

    def zero_row(r, _):
        for j in range(DIM // LANES):
            rows_bufs[0][r, pl.ds(j * LANES, LANES)] = (
                jnp.zeros((LANES,), jnp.float32))
        return 0
    lax.fori_loop(0, STRIPE, zero_row, 0)
    for k in range(STRIPES_PER_TILE):
        s = sid + k * NS
        @pl.when(s < N_STRIPES)
        def _():
            pltpu.sync_copy(rows_bufs[0].at[pl.ds(0, STRIPE)],
                            acc_sh.at[pl.ds(s * STRIPE, STRIPE)])
    plsc.subcore_barrier()

    pltpu.make_async_copy(
        src_hbm.at[pl.ds(0, CHUNK)], src_bufs[0], isems[0]).wait()
    start_gather(0)

    def ring_body(g, _):
        i0 = g * NBUF
        for j in range(NBUF):
            body_fn(i0 + j, j, (j + 1) % NBUF)
        return 0

    lax.fori_loop(0, N_RING, ring_body, 0)
    wait_scatter(0)
    wait_scatter(1)
    tbase = base0 + N_CHUNKS * CHUNK
    pltpu.async_copy(src_hbm.at[pl.ds(tbase, TAIL)],
                     src_bufs[0].at[pl.ds(0, TAIL)], isems[0])
    pltpu.async_copy(dst_hbm.at[pl.ds(tbase, TAIL)], tdst_buf, dsems[0])
    pltpu.async_copy(ea_hbm.at[pl.ds(tbase, TAIL)],
                     msg_bufs[0].at[pl.ds(0, TAIL)], esems[0])
    pltpu.make_async_copy(src_hbm.at[pl.ds(0, TAIL)],
                          src_bufs[0].at[pl.ds(0, TAIL)], isems[0]).wait()
    pltpu.async_copy(x_hbm.at[src_bufs[0].at[pl.ds(0, TAIL)]],
                     rows_bufs[0].at[pl.ds(0, TAIL)], gsems[0])
    pltpu.make_async_copy(x_hbm.at[src_bufs[0].at[pl.ds(0, TAIL)]],
                          rows_bufs[0].at[pl.ds(0, TAIL)], gsems[0]).wait()
    pltpu.make_async_copy(ea_hbm.at[pl.ds(0, TAIL)],
                          msg_bufs[0].at[pl.ds(0, TAIL)], esems[0]).wait()

    def tail_row(r, _):
        for j in range(DIM // LANES):
            sl = pl.ds(j * LANES, LANES)
            msg_bufs[0][r, sl] = jnp.maximum(
                msg_bufs[0][r, sl] + rows_bufs[0][r, sl], 0.0)
        return 0
    lax.fori_loop(0, TAIL, tail_row, 0)
    pltpu.make_async_copy(dst_hbm.at[pl.ds(0, TAIL)], tdst_buf,
                          dsems[0]).wait()
    pltpu.async_copy(msg_bufs[0].at[pl.ds(0, TAIL)], acc_sh.at[tdst_buf],
                     ssems[0], add=True)
    pltpu.make_async_copy(msg_bufs[0].at[pl.ds(0, TAIL)],
                          acc_sh.at[tdst_buf], ssems[0]).wait()
    wait_scatter(2)
    wait_scatter(3)
    plsc.subcore_barrier()

    for k in range(STRIPES_PER_TILE):
        s = sid + k * NS
        @pl.when(s < N_STRIPES)
        def _():
            pltpu.sync_copy(acc_sh.at[pl.ds(s * STRIPE, STRIPE)],
                            out_hbm.at[cid, pl.ds(s * STRIPE, STRIPE)])


_sc_edge = functools.partial(
    pl.kernel,
    out_type=jax.ShapeDtypeStruct((NC, N_NODES, DIM), jnp.float32),
    mesh=plsc.VectorSubcoreMesh(core_axis_name="c", subcore_axis_name="s",
                                num_cores=NC, num_subcores=NS),
    scratch_types=[
        tuple(pltpu.VMEM((CHUNK,), jnp.int32) for _ in range(NBUF)),
        tuple(pltpu.VMEM((SUB,), jnp.int32) for _ in range(NBUF)),
        tuple(pltpu.VMEM((SUB,), jnp.int32) for _ in range(NBUF)),
        tuple(pltpu.VMEM((CHUNK, DIM), jnp.float32) for _ in range(NBUF)),
        tuple(pltpu.VMEM((CHUNK, DIM), jnp.float32) for _ in range(NBUF)),
        pltpu.VMEM((TAIL,), jnp.int32),
        pltpu.VMEM_SHARED((N_NODES, DIM), jnp.float32),
        tuple(pltpu.SemaphoreType.DMA for _ in range(NBUF)),
        tuple(pltpu.SemaphoreType.DMA for _ in range(NBUF)),
        tuple(pltpu.SemaphoreType.DMA for _ in range(NBUF)),
        tuple(pltpu.SemaphoreType.DMA for _ in range(NBUF)),
        tuple(pltpu.SemaphoreType.DMA for _ in range(4)),
    ],
)(_sc_edge_body)


def _tc_mlp_body(eps_ref, x_ref, p_ref, w1_ref, b1_ref, g_ref, be_ref,
                 w2_ref, b2_ref, o_ref):
    h = x_ref[...] * (1.0 + eps_ref[0]) + p_ref[0] + p_ref[1]
    h1 = lax.dot_general(h, w1_ref[...], (((1,), (1,)), ((), ())),
                         preferred_element_type=jnp.float32) + b1_ref[...]
    mean = jnp.mean(h1, axis=0, keepdims=True)
    var = jnp.mean(jnp.square(h1 - mean), axis=0, keepdims=True)
    h2 = (h1 - mean) * lax.rsqrt(var + 1e-5) * g_ref[...] + be_ref[...]
    h2 = jnp.maximum(h2, 0.0)
    o_ref[...] = lax.dot_general(h2, w2_ref[...], (((1,), (1,)), ((), ())),
                                 preferred_element_type=jnp.float32) + b2_ref[...]


def _tc_mlp(eps, x, partials, w1, b1, gamma, beta, w2, b2):
    return pl.pallas_call(
        _tc_mlp_body,
        out_shape=jax.ShapeDtypeStruct((N_NODES, DIM), jnp.float32),
        in_specs=[
            pl.BlockSpec(memory_space=pltpu.SMEM),
            pl.BlockSpec(memory_space=pltpu.VMEM),
            pl.BlockSpec(memory_space=pltpu.VMEM),
            pl.BlockSpec(memory_space=pltpu.VMEM),
            pl.BlockSpec(memory_space=pltpu.VMEM),
            pl.BlockSpec(memory_space=pltpu.VMEM),
            pl.BlockSpec(memory_space=pltpu.VMEM),
            pl.BlockSpec(memory_space=pltpu.VMEM),
            pl.BlockSpec(memory_space=pltpu.VMEM),
        ],
        out_specs=pl.BlockSpec(memory_space=pltpu.VMEM),
    )(eps, x, partials, w1, b1, gamma, beta, w2, b2)


def kernel(x, edge_index, edge_attr, eps, W1, b1, gamma, beta, W2, b2):
    dst = edge_index[0]
    src = edge_index[1]
    partials = _sc_edge(x, src, dst, edge_attr)
    return _tc_mlp(eps, x, partials, W1,
                   b1.reshape(1, DIM), gamma.reshape(1, DIM),
                   beta.reshape(1, DIM), W2, b2.reshape(1, DIM))


# speedup vs baseline: 1.6885x; 1.0143x over previous
import functools

import jax
import jax.numpy as jnp
from jax import lax
from jax.experimental import pallas as pl
from jax.experimental.pallas import tpu as pltpu
from jax.experimental.pallas import tpu_sc as plsc

N_NODES = 10000
N_EDGES = 320000
DIM = 128

NC = 2
NS = 16
NW = NC * NS
E_PER_W = N_EDGES // NW
CHUNK = 96
N_CHUNKS = E_PER_W // CHUNK
TAIL = E_PER_W - N_CHUNKS * CHUNK
STRIPE = 80
N_STRIPES = N_NODES // STRIPE
STRIPES_PER_TILE = -(-N_STRIPES // NS)
LANES = 16


NBUF = 2
N_RING = N_CHUNKS // NBUF


SUB = CHUNK // 2


def _sc_edge_body(x_hbm, comb_hbm, ea_hbm, out_hbm,
                  comb_bufs, src_bufs, dst_bufs, rows_bufs, msg_bufs,
                  tdst_buf, acc_sh, isems, gsems, esems, ssems):
    cid = lax.axis_index("c")
    sid = lax.axis_index("s")
    wid = sid * NC + cid
    base0 = wid * E_PER_W

    def start_comb(i, b):
        base = base0 + i * CHUNK
        pltpu.async_copy(comb_hbm.at[pl.ds(base, CHUNK)], comb_bufs[b],
                         isems[b])

    def start_ea(i, b):
        base = base0 + i * CHUNK
        pltpu.async_copy(ea_hbm.at[pl.ds(base, CHUNK)], msg_bufs[b], esems[b])

    def start_gather(b):
        pltpu.async_copy(x_hbm.at[src_bufs[b]], rows_bufs[b], gsems[b])

    def unpack_src(b):
        for k in range(CHUNK // LANES):
            sl = pl.ds(k * LANES, LANES)
            src_bufs[b][sl] = lax.shift_right_logical(comb_bufs[b][sl], 14)

    def unpack_dst(b):
        for k in range(CHUNK // LANES):
            sl = pl.ds(k * LANES, LANES)
            dst_bufs[b][sl] = lax.bitwise_and(
                comb_bufs[b][sl], jnp.full((LANES,), 16383, jnp.int32))

    def wait_scatter(b):
        pltpu.make_async_copy(
            msg_bufs[b], acc_sh.at[dst_bufs[b]], ssems[b]).wait()

    def body_fn(i, b, bn):
        pltpu.make_async_copy(
            x_hbm.at[src_bufs[b]], rows_bufs[b], gsems[b]).wait()
        pltpu.make_async_copy(
            ea_hbm.at[pl.ds(0, CHUNK)], msg_bufs[b], esems[b]).wait()

        @pl.when(i + 2 < N_CHUNKS)
        def _():
            start_comb(i + 2, b)

        @pl.when(i + 1 < N_CHUNKS)
        def _():
            pltpu.make_async_copy(
                comb_hbm.at[pl.ds(0, CHUNK)], comb_bufs[bn], isems[bn]).wait()
            unpack_src(bn)
            start_gather(bn)
            @pl.when(i > 0)
            def _():
                wait_scatter(bn)
            unpack_dst(bn)
            start_ea(i + 1, bn)

        def row_body(r, _):
            for j in range(DIM // LANES):
                sl = pl.ds(j * LANES, LANES)
                msg_bufs[b][r, sl] = jnp.maximum(
                    msg_bufs[b][r, sl] + rows_bufs[b][r, sl], 0.0)
            return 0
        lax.fori_loop(0, CHUNK, row_body, 0)
        pltpu.async_copy(msg_bufs[b], acc_sh.at[dst_bufs[b]], ssems[b],
                         add=True)

    start_comb(0, 0)
    start_ea(0, 0)
    start_comb(1, 1)

    def zero_row(r, _):
        for j in range(DIM // LANES):
            rows_bufs[0][r, pl.ds(j * LANES, LANES)] = (
                jnp.zeros((LANES,), jnp.float32))
        return 0
    lax.fori_loop(0, STRIPE, zero_row, 0)
    for k in range(STRIPES_PER_TILE):
        s = sid + k * NS
        @pl.when(s < N_STRIPES)
        def _():
            pltpu.sync_copy(rows_bufs[0].at[pl.ds(0, STRIPE)],
                            acc_sh.at[pl.ds(s * STRIPE, STRIPE)])
    plsc.subcore_barrier()

    pltpu.make_async_copy(
        comb_hbm.at[pl.ds(0, CHUNK)], comb_bufs[0], isems[0]).wait()
    unpack_src(0)
    unpack_dst(0)
    start_gather(0)

    def ring_body(g, _):
        i0 = g * NBUF
        for j in range(NBUF):
            body_fn(i0 + j, j, (j + 1) % NBUF)
        return 0

    lax.fori_loop(0, N_RING, ring_body, 0)
    wait_scatter(0)
    tbase = base0 + N_CHUNKS * CHUNK
    pltpu.async_copy(comb_hbm.at[pl.ds(tbase, TAIL)],
                     comb_bufs[0].at[pl.ds(0, TAIL)], isems[0])
    pltpu.async_copy(ea_hbm.at[pl.ds(tbase, TAIL)],
                     msg_bufs[0].at[pl.ds(0, TAIL)], esems[0])
    pltpu.make_async_copy(comb_hbm.at[pl.ds(0, TAIL)],
                          comb_bufs[0].at[pl.ds(0, TAIL)], isems[0]).wait()
    tsl = pl.ds(0, TAIL)
    src_bufs[0][tsl] = lax.shift_right_logical(comb_bufs[0][tsl], 14)
    tdst_buf[tsl] = lax.bitwise_and(
        comb_bufs[0][tsl], jnp.full((TAIL,), 16383, jnp.int32))
    pltpu.async_copy(x_hbm.at[src_bufs[0].at[pl.ds(0, TAIL)]],
                     rows_bufs[0].at[pl.ds(0, TAIL)], gsems[0])
    pltpu.make_async_copy(x_hbm.at[src_bufs[0].at[pl.ds(0, TAIL)]],
                          rows_bufs[0].at[pl.ds(0, TAIL)], gsems[0]).wait()
    pltpu.make_async_copy(ea_hbm.at[pl.ds(0, TAIL)],
                          msg_bufs[0].at[pl.ds(0, TAIL)], esems[0]).wait()

    def tail_row(r, _):
        for j in range(DIM // LANES):
            sl = pl.ds(j * LANES, LANES)
            msg_bufs[0][r, sl] = jnp.maximum(
                msg_bufs[0][r, sl] + rows_bufs[0][r, sl], 0.0)
        return 0
    lax.fori_loop(0, TAIL, tail_row, 0)
    pltpu.async_copy(msg_bufs[0].at[pl.ds(0, TAIL)], acc_sh.at[tdst_buf],
                     ssems[0], add=True)
    pltpu.make_async_copy(msg_bufs[0].at[pl.ds(0, TAIL)],
                          acc_sh.at[tdst_buf], ssems[0]).wait()
    wait_scatter(1)
    plsc.subcore_barrier()

    for k in range(STRIPES_PER_TILE):
        s = sid + k * NS
        @pl.when(s < N_STRIPES)
        def _():
            pltpu.sync_copy(acc_sh.at[pl.ds(s * STRIPE, STRIPE)],
                            out_hbm.at[cid, pl.ds(s * STRIPE, STRIPE)])


_sc_edge = functools.partial(
    pl.kernel,
    out_type=jax.ShapeDtypeStruct((NC, N_NODES, DIM), jnp.float32),
    mesh=plsc.VectorSubcoreMesh(core_axis_name="c", subcore_axis_name="s",
                                num_cores=NC, num_subcores=NS),
    scratch_types=[
        tuple(pltpu.VMEM((CHUNK,), jnp.int32) for _ in range(NBUF)),
        tuple(pltpu.VMEM((CHUNK,), jnp.int32) for _ in range(NBUF)),
        tuple(pltpu.VMEM((CHUNK,), jnp.int32) for _ in range(NBUF)),
        tuple(pltpu.VMEM((CHUNK, DIM), jnp.float32) for _ in range(NBUF)),
        tuple(pltpu.VMEM((CHUNK, DIM), jnp.float32) for _ in range(NBUF)),
        pltpu.VMEM((TAIL,), jnp.int32),
        pltpu.VMEM_SHARED((N_NODES, DIM), jnp.float32),
        tuple(pltpu.SemaphoreType.DMA for _ in range(NBUF)),
        tuple(pltpu.SemaphoreType.DMA for _ in range(NBUF)),
        tuple(pltpu.SemaphoreType.DMA for _ in range(NBUF)),
        tuple(pltpu.SemaphoreType.DMA for _ in range(NBUF)),
    ],
)(_sc_edge_body)


def _tc_mlp_body(eps_ref, x_ref, p_ref, w1_ref, b1_ref, g_ref, be_ref,
                 w2_ref, b2_ref, o_ref):
    h = x_ref[...] * (1.0 + eps_ref[0]) + p_ref[0] + p_ref[1]
    h1 = lax.dot_general(h, w1_ref[...], (((1,), (1,)), ((), ())),
                         preferred_element_type=jnp.float32) + b1_ref[...]
    mean = jnp.mean(h1, axis=0, keepdims=True)
    var = jnp.mean(jnp.square(h1 - mean), axis=0, keepdims=True)
    h2 = (h1 - mean) * lax.rsqrt(var + 1e-5) * g_ref[...] + be_ref[...]
    h2 = jnp.maximum(h2, 0.0)
    o_ref[...] = lax.dot_general(h2, w2_ref[...], (((1,), (1,)), ((), ())),
                                 preferred_element_type=jnp.float32) + b2_ref[...]


def _tc_mlp(eps, x, partials, w1, b1, gamma, beta, w2, b2):
    return pl.pallas_call(
        _tc_mlp_body,
        out_shape=jax.ShapeDtypeStruct((N_NODES, DIM), jnp.float32),
        in_specs=[
            pl.BlockSpec(memory_space=pltpu.SMEM),
            pl.BlockSpec(memory_space=pltpu.VMEM),
            pl.BlockSpec(memory_space=pltpu.VMEM),
            pl.BlockSpec(memory_space=pltpu.VMEM),
            pl.BlockSpec(memory_space=pltpu.VMEM),
            pl.BlockSpec(memory_space=pltpu.VMEM),
            pl.BlockSpec(memory_space=pltpu.VMEM),
            pl.BlockSpec(memory_space=pltpu.VMEM),
            pl.BlockSpec(memory_space=pltpu.VMEM),
        ],
        out_specs=pl.BlockSpec(memory_space=pltpu.VMEM),
    )(eps, x, partials, w1, b1, gamma, beta, w2, b2)


def kernel(x, edge_index, edge_attr, eps, W1, b1, gamma, beta, W2, b2):
    dst = edge_index[0]
    src = edge_index[1]
    comb = jnp.bitwise_or(jnp.left_shift(src, 14), dst)
    partials = _sc_edge(x, comb, edge_attr)
    return _tc_mlp(eps, x, partials, W1,
                   b1.reshape(1, DIM), gamma.reshape(1, DIM),
                   beta.reshape(1, DIM), W2, b2.reshape(1, DIM))
